# baseline (speedup 1.0000x reference)
import jax
import jax.numpy as jnp
from jax.experimental import pallas as pl

B, C, H, W_SPATIAL = 1024, 768, 14, 14
HW = H * W_SPATIAL
K_TOTAL = C * HW
NUM_CLUSTER = 10
BLOCK_B = 128
BLOCK_K = K_TOTAL // 8
GRID_I = B // BLOCK_B
GRID_K = K_TOTAL // BLOCK_K


def _sem_head_kernel(x_ref, w_ref, b_ref, out_ref):
    k = pl.program_id(1)

    @pl.when(k == 0)
    def _init():
        out_ref[...] = jnp.broadcast_to(b_ref[...], (BLOCK_B, NUM_CLUSTER))

    xb = x_ref[...].astype(jnp.bfloat16)
    acc = jax.lax.dot_general(
        xb, w_ref[...],
        dimension_numbers=(((1,), (1,)), ((), ())),
        preferred_element_type=jnp.float32,
    )
    out_ref[...] += acc


@jax.jit
def kernel(fea, W, b):
    fea_flat = fea.reshape(B, K_TOTAL)
    w2t = jnp.repeat(W * (1.0 / HW), HW, axis=1).astype(jnp.bfloat16)
    b2 = b.reshape(1, NUM_CLUSTER)
    return pl.pallas_call(
        _sem_head_kernel,
        grid=(GRID_I, GRID_K),
        in_specs=[
            pl.BlockSpec((BLOCK_B, BLOCK_K), lambda i, k: (i, k)),
            pl.BlockSpec((NUM_CLUSTER, BLOCK_K), lambda i, k: (0, k)),
            pl.BlockSpec((1, NUM_CLUSTER), lambda i, k: (0, 0)),
        ],
        out_specs=pl.BlockSpec((BLOCK_B, NUM_CLUSTER), lambda i, k: (i, 0)),
        out_shape=jax.ShapeDtypeStruct((B, NUM_CLUSTER), jnp.float32),
    )(fea_flat, w2t, b2)
